# baseline (speedup 1.0000x reference)
import jax
import jax.numpy as jnp
from jax.experimental import pallas as pl

NUM_Q = 8
K = 1024
D = 256
BM = 1152


def _rvq_kernel(x_ref, cb_ref, idx_ref, quant_ref):
    r = x_ref[...]
    acc = jnp.zeros_like(r)
    iota_k = jax.lax.broadcasted_iota(jnp.int32, (BM, K), 1)
    for i in range(NUM_Q):
        embed = cb_ref[i]
        norms = jnp.sum(embed * embed, axis=1)
        xx = jnp.sum(r * r, axis=1, keepdims=True)
        mm = jax.lax.dot_general(
            r.astype(jnp.bfloat16), embed.astype(jnp.bfloat16),
            (((1,), (1,)), ((), ())),
            preferred_element_type=jnp.float32)
        scores = -(xx - 2.0 * mm + norms[None, :])
        m = jnp.max(scores, axis=1, keepdims=True)
        idx = jnp.min(jnp.where(scores == m, iota_k, K), axis=1)
        onehot = (iota_k == idx[:, None]).astype(jnp.float32)
        q = jax.lax.dot_general(
            onehot, embed, (((1,), (0,)), ((), ())),
            preferred_element_type=jnp.float32,
            precision=jax.lax.Precision.HIGHEST)
        r = r - q
        acc = acc + q
        idx_ref[i, :] = idx
    quant_ref[...] = acc


@jax.jit
def kernel(x, codebooks):
    shape = x.shape
    n = shape[0] * shape[1]
    x_flat = x.reshape(n, D)
    grid = n // BM
    indices, quant = pl.pallas_call(
        _rvq_kernel,
        grid=(grid,),
        in_specs=[
            pl.BlockSpec((BM, D), lambda i: (i, 0)),
            pl.BlockSpec((NUM_Q, K, D), lambda i: (0, 0, 0)),
        ],
        out_specs=[
            pl.BlockSpec((NUM_Q, BM), lambda i: (0, i)),
            pl.BlockSpec((BM, D), lambda i: (i, 0)),
        ],
        out_shape=[
            jax.ShapeDtypeStruct((NUM_Q, n), jnp.int32),
            jax.ShapeDtypeStruct((n, D), jnp.float32),
        ],
    )(x_flat, codebooks)
    return indices.reshape(NUM_Q, shape[0], shape[1]), quant.reshape(shape)


# speedup vs baseline: 1.5281x; 1.5281x over previous
import jax
import jax.numpy as jnp
from jax.experimental import pallas as pl

NUM_Q = 8
K = 1024
D = 256
BM = 1152


def _rvq_kernel(x_ref, cb_ref, idx_ref, quant_ref):
    r = x_ref[...]
    acc = jnp.zeros_like(r)
    iota_k = jax.lax.broadcasted_iota(jnp.int32, (BM, K), 1)
    for i in range(NUM_Q):
        embed = cb_ref[i]
        norms = jnp.sum(embed * embed, axis=1)
        xx = jnp.sum(r * r, axis=1, keepdims=True)
        mm = jax.lax.dot_general(
            r.astype(jnp.bfloat16), embed.astype(jnp.bfloat16),
            (((1,), (1,)), ((), ())),
            preferred_element_type=jnp.float32)
        scores = -(xx - 2.0 * mm + norms[None, :])
        m = jnp.max(scores, axis=1, keepdims=True)
        idx = jnp.min(jnp.where(scores == m, iota_k, K), axis=1)
        onehot = (iota_k == idx[:, None]).astype(jnp.bfloat16)
        e_hi = embed.astype(jnp.bfloat16)
        r1 = embed - e_hi.astype(jnp.float32)
        e_mid = r1.astype(jnp.bfloat16)
        e_rem = (r1 - e_mid.astype(jnp.float32)).astype(jnp.bfloat16)
        dims = (((1,), (0,)), ((), ()))
        q_hi = jax.lax.dot_general(onehot, e_hi, dims,
                                   preferred_element_type=jnp.float32)
        q_mid = jax.lax.dot_general(onehot, e_mid, dims,
                                    preferred_element_type=jnp.float32)
        q_rem = jax.lax.dot_general(onehot, e_rem, dims,
                                    preferred_element_type=jnp.float32)
        q = (q_hi + q_mid) + q_rem
        r = r - q
        acc = acc + q
        idx_ref[i, :] = idx
    quant_ref[...] = acc


@jax.jit
def kernel(x, codebooks):
    shape = x.shape
    n = shape[0] * shape[1]
    x_flat = x.reshape(n, D)
    grid = n // BM
    indices, quant = pl.pallas_call(
        _rvq_kernel,
        grid=(grid,),
        in_specs=[
            pl.BlockSpec((BM, D), lambda i: (i, 0)),
            pl.BlockSpec((NUM_Q, K, D), lambda i: (0, 0, 0)),
        ],
        out_specs=[
            pl.BlockSpec((NUM_Q, BM), lambda i: (0, i)),
            pl.BlockSpec((BM, D), lambda i: (i, 0)),
        ],
        out_shape=[
            jax.ShapeDtypeStruct((NUM_Q, n), jnp.int32),
            jax.ShapeDtypeStruct((n, D), jnp.float32),
        ],
    )(x_flat, codebooks)
    return indices.reshape(NUM_Q, shape[0], shape[1]), quant.reshape(shape)


# speedup vs baseline: 1.5784x; 1.0329x over previous
import jax
import jax.numpy as jnp
from jax.experimental import pallas as pl

NUM_Q = 8
K = 1024
D = 256
BM = 1152


def _stage_kernel(r_ref, xx_ref, cb_ref, idx_ref, rout_ref):
    r = r_ref[...]
    xx = xx_ref[...]
    embed = cb_ref[...]
    norms = jnp.sum(embed * embed, axis=1)
    mm = jax.lax.dot_general(
        r.astype(jnp.bfloat16), embed.astype(jnp.bfloat16),
        (((1,), (1,)), ((), ())),
        preferred_element_type=jnp.float32)
    scores = -(xx - 2.0 * mm + norms[None, :])
    m = jnp.max(scores, axis=1, keepdims=True)
    iota_k = jax.lax.broadcasted_iota(jnp.int32, (BM, K), 1)
    idx = jnp.min(jnp.where(scores == m, iota_k, K), axis=1)
    onehot = (iota_k == idx[:, None]).astype(jnp.bfloat16)
    e_hi = embed.astype(jnp.bfloat16)
    r1 = embed - e_hi.astype(jnp.float32)
    e_mid = r1.astype(jnp.bfloat16)
    e_rem = (r1 - e_mid.astype(jnp.float32)).astype(jnp.bfloat16)
    dims = (((1,), (0,)), ((), ()))
    q_hi = jax.lax.dot_general(onehot, e_hi, dims,
                               preferred_element_type=jnp.float32)
    q_mid = jax.lax.dot_general(onehot, e_mid, dims,
                                preferred_element_type=jnp.float32)
    q_rem = jax.lax.dot_general(onehot, e_rem, dims,
                                preferred_element_type=jnp.float32)
    q = (q_hi + q_mid) + q_rem
    rout_ref[...] = r - q
    idx_ref[...] = idx[:, None]


def _last_stage_kernel(r_ref, xx_ref, cb_ref, x0_ref, idx_ref, rout_ref,
                       quant_ref):
    _stage_kernel(r_ref, xx_ref, cb_ref, idx_ref, rout_ref)
    quant_ref[...] = x0_ref[...] - rout_ref[...]


def _make_stage(n, last):
    in_specs = [
        pl.BlockSpec((BM, D), lambda b: (b, 0)),
        pl.BlockSpec((BM, 1), lambda b: (b, 0)),
        pl.BlockSpec((K, D), lambda b: (0, 0)),
    ]
    out_specs = [
        pl.BlockSpec((BM, 1), lambda b: (b, 0)),
        pl.BlockSpec((BM, D), lambda b: (b, 0)),
    ]
    out_shape = [
        jax.ShapeDtypeStruct((n, 1), jnp.int32),
        jax.ShapeDtypeStruct((n, D), jnp.float32),
    ]
    if last:
        in_specs.append(pl.BlockSpec((BM, D), lambda b: (b, 0)))
        out_specs.append(pl.BlockSpec((BM, D), lambda b: (b, 0)))
        out_shape.append(jax.ShapeDtypeStruct((n, D), jnp.float32))
    return pl.pallas_call(
        _last_stage_kernel if last else _stage_kernel,
        grid=(n // BM,),
        in_specs=in_specs,
        out_specs=out_specs,
        out_shape=out_shape,
    )


@jax.jit
def kernel(x, codebooks):
    shape = x.shape
    n = shape[0] * shape[1]
    x_flat = x.reshape(n, D)
    stage = _make_stage(n, last=False)
    last_stage = _make_stage(n, last=True)
    r = x_flat
    idxs = []
    quant = None
    for i in range(NUM_Q):
        xx = jnp.sum(r * r, axis=1, keepdims=True)
        if i < NUM_Q - 1:
            idx_i, r = stage(r, xx, codebooks[i])
        else:
            idx_i, r, quant = last_stage(r, xx, codebooks[i], x_flat)
        idxs.append(idx_i[:, 0])
    indices = jnp.stack(idxs).reshape(NUM_Q, shape[0], shape[1])
    return indices, quant.reshape(shape)


# speedup vs baseline: 1.8341x; 1.1620x over previous
import jax
import jax.numpy as jnp
from jax.experimental import pallas as pl
from jax.experimental.pallas import tpu as pltpu

NUM_Q = 8
K = 1024
D = 256
BM = 1152
HM = BM // 2


def _half(r, xx, hi_s, mid_s, rem_s, norms_s):
    mm = jax.lax.dot_general(
        r.astype(jnp.bfloat16), hi_s[...], (((1,), (1,)), ((), ())),
        preferred_element_type=jnp.float32)
    scores = -(xx - 2.0 * mm + norms_s[...])
    m = jnp.max(scores, axis=1, keepdims=True)
    iota_k = jax.lax.broadcasted_iota(jnp.int32, (HM, K), 1)
    idx = jnp.min(jnp.where(scores == m, iota_k, K), axis=1)
    onehot = (iota_k == idx[:, None]).astype(jnp.bfloat16)
    dims = (((1,), (0,)), ((), ()))
    q_hi = jax.lax.dot_general(onehot, hi_s[...], dims,
                               preferred_element_type=jnp.float32)
    q_mid = jax.lax.dot_general(onehot, mid_s[...], dims,
                                preferred_element_type=jnp.float32)
    q_rem = jax.lax.dot_general(onehot, rem_s[...], dims,
                                preferred_element_type=jnp.float32)
    q = (q_hi + q_mid) + q_rem
    return idx, r - q


def _stage_kernel(r_ref, xx_ref, cb_ref, idx_ref, rout_ref,
                  hi_s, mid_s, rem_s, norms_s):
    @pl.when(pl.program_id(0) == 0)
    def _():
        embed = cb_ref[...]
        norms_s[...] = jnp.sum(embed * embed, axis=1)[None, :]
        e_hi = embed.astype(jnp.bfloat16)
        r1 = embed - e_hi.astype(jnp.float32)
        e_mid = r1.astype(jnp.bfloat16)
        hi_s[...] = e_hi
        mid_s[...] = e_mid
        rem_s[...] = (r1 - e_mid.astype(jnp.float32)).astype(jnp.bfloat16)

    idx_a, rout_a = _half(r_ref[:HM, :], xx_ref[:HM, :],
                          hi_s, mid_s, rem_s, norms_s)
    idx_b, rout_b = _half(r_ref[HM:, :], xx_ref[HM:, :],
                          hi_s, mid_s, rem_s, norms_s)
    idx_ref[:HM, :] = idx_a[:, None]
    idx_ref[HM:, :] = idx_b[:, None]
    rout_ref[:HM, :] = rout_a
    rout_ref[HM:, :] = rout_b


def _last_stage_kernel(r_ref, xx_ref, cb_ref, x0_ref, idx_ref, rout_ref,
                       quant_ref, hi_s, mid_s, rem_s, norms_s):
    _stage_kernel(r_ref, xx_ref, cb_ref, idx_ref, rout_ref,
                  hi_s, mid_s, rem_s, norms_s)
    quant_ref[...] = x0_ref[...] - rout_ref[...]


def _make_stage(n, last):
    in_specs = [
        pl.BlockSpec((BM, D), lambda b: (b, 0)),
        pl.BlockSpec((BM, 1), lambda b: (b, 0)),
        pl.BlockSpec((K, D), lambda b: (0, 0)),
    ]
    out_specs = [
        pl.BlockSpec((BM, 1), lambda b: (b, 0)),
        pl.BlockSpec((BM, D), lambda b: (b, 0)),
    ]
    out_shape = [
        jax.ShapeDtypeStruct((n, 1), jnp.int32),
        jax.ShapeDtypeStruct((n, D), jnp.float32),
    ]
    if last:
        in_specs.append(pl.BlockSpec((BM, D), lambda b: (b, 0)))
        out_specs.append(pl.BlockSpec((BM, D), lambda b: (b, 0)))
        out_shape.append(jax.ShapeDtypeStruct((n, D), jnp.float32))
    return pl.pallas_call(
        _last_stage_kernel if last else _stage_kernel,
        grid=(n // BM,),
        in_specs=in_specs,
        out_specs=out_specs,
        out_shape=out_shape,
        scratch_shapes=[
            pltpu.VMEM((K, D), jnp.bfloat16),
            pltpu.VMEM((K, D), jnp.bfloat16),
            pltpu.VMEM((K, D), jnp.bfloat16),
            pltpu.VMEM((1, K), jnp.float32),
        ],
    )


@jax.jit
def kernel(x, codebooks):
    shape = x.shape
    n = shape[0] * shape[1]
    x_flat = x.reshape(n, D)
    stage = _make_stage(n, last=False)
    last_stage = _make_stage(n, last=True)
    r = x_flat
    idxs = []
    quant = None
    for i in range(NUM_Q):
        xx = jnp.sum(r * r, axis=1, keepdims=True)
        if i < NUM_Q - 1:
            idx_i, r = stage(r, xx, codebooks[i])
        else:
            idx_i, r, quant = last_stage(r, xx, codebooks[i], x_flat)
        idxs.append(idx_i[:, 0])
    indices = jnp.stack(idxs).reshape(NUM_Q, shape[0], shape[1])
    return indices, quant.reshape(shape)
